# baseline (speedup 1.0000x reference)
import functools

import jax
import jax.numpy as jnp
from jax import lax
from jax.experimental import pallas as pl
from jax.experimental.pallas import tpu as pltpu
from jax.experimental.pallas import tpu_sc as plsc

BATCH = 4096
SEQ_LEN = 200
EMBEDDING_DIM = 32

NUM_WORKERS = 32
SEQS_PER_W = BATCH // NUM_WORKERS
NSEQ = 8
N_CHUNK = SEQS_PER_W // NSEQ
SPLITS = ((0, 104), (104, 96))


def _gather_kernel(idx_hbm, table_hbm, out_hbm, idx_v, rows_v, g0, g1, w0, w1):
    gsems = (g0, g1)
    wsems = (w0, w1)
    c = lax.axis_index("c")
    s = lax.axis_index("s")
    wid = s * 2 + c
    seq_base = wid * SEQS_PER_W

    def issue_gathers(chunk, b):
        pltpu.sync_copy(idx_hbm.at[pl.ds(seq_base + chunk * NSEQ, NSEQ)], idx_v.at[b])
        for q in range(NSEQ):
            for off, size in SPLITS:
                pltpu.async_copy(
                    table_hbm.at[idx_v.at[b, q, pl.ds(off, size)]],
                    rows_v.at[b, q, pl.ds(off, size)],
                    gsems[b],
                )

    def drain_gathers(b):
        pltpu.make_async_copy(out_hbm.at[pl.ds(0, NSEQ)], rows_v.at[b], gsems[b]).wait()

    def start_write(chunk, b):
        pltpu.async_copy(
            rows_v.at[b],
            out_hbm.at[pl.ds(seq_base + chunk * NSEQ, NSEQ)],
            wsems[b],
        )

    def drain_write(b):
        pltpu.make_async_copy(out_hbm.at[pl.ds(0, NSEQ)], rows_v.at[b], wsems[b]).wait()

    issue_gathers(0, 0)

    def body(outer, carry):
        @pl.when(outer >= 1)
        def _():
            drain_write(1)

        issue_gathers(2 * outer + 1, 1)
        drain_gathers(0)
        start_write(2 * outer, 0)

        @pl.when(outer + 1 < N_CHUNK // 2)
        def _():
            drain_write(0)
            issue_gathers(2 * outer + 2, 0)

        drain_gathers(1)
        start_write(2 * outer + 1, 1)
        return carry

    lax.fori_loop(0, N_CHUNK // 2, body, 0)
    drain_write(0)
    drain_write(1)


def kernel(input, weight):
    mesh = plsc.VectorSubcoreMesh(core_axis_name="c", subcore_axis_name="s")
    run = functools.partial(
        pl.kernel,
        mesh=mesh,
        out_type=jax.ShapeDtypeStruct((BATCH, SEQ_LEN, EMBEDDING_DIM), jnp.float32),
        scratch_types=[
            pltpu.VMEM((2, NSEQ, SEQ_LEN), jnp.int32),
            pltpu.VMEM((2, NSEQ, SEQ_LEN, EMBEDDING_DIM), jnp.float32),
            pltpu.SemaphoreType.DMA,
            pltpu.SemaphoreType.DMA,
            pltpu.SemaphoreType.DMA,
            pltpu.SemaphoreType.DMA,
        ],
        compiler_params=pltpu.CompilerParams(use_tc_tiling_on_sc=False),
    )(_gather_kernel)
    return run(input, weight)
